# baseline (speedup 1.0000x reference)
import functools

import jax
import jax.numpy as jnp
from jax import lax
from jax.experimental import pallas as pl
from jax.experimental.pallas import tpu as pltpu
from jax.experimental.pallas import tpu_sc as plsc

NC = 2
NS = 16
CHUNK = 128
SEG = 40
NBUF = 2
D = 128


def _mesh():
    return plsc.VectorSubcoreMesh(
        core_axis_name="c", subcore_axis_name="s",
        num_cores=NC, num_subcores=NS)


def _mm_body(h_ref, w_ref, norm_ref, out_ref):
    out_ref[...] = (
        jnp.dot(h_ref[...], w_ref[...], preferred_element_type=jnp.float32)
        * norm_ref[...]
    )


def _matmul_norm(h, W, norm):
    n, d_in = h.shape
    d_out = W.shape[1]
    bm = 2000
    return pl.pallas_call(
        _mm_body,
        grid=(n // bm,),
        in_specs=[
            pl.BlockSpec((bm, d_in), lambda i: (i, 0)),
            pl.BlockSpec((d_in, d_out), lambda i: (0, 0)),
            pl.BlockSpec((bm, 1), lambda i: (i, 0)),
        ],
        out_specs=pl.BlockSpec((bm, d_out), lambda i: (i, 0)),
        out_shape=jax.ShapeDtypeStruct((n, d_out), jnp.float32),
    )(h, W, norm)


def _combine_body(a_ref, b_ref, norm_ref, out_ref):
    out_ref[...] = (a_ref[0] + b_ref[0]) * norm_ref[...]


def _combine(partials, norm):
    n, _ = norm.shape
    bm = 2000
    return pl.pallas_call(
        _combine_body,
        grid=(n // bm,),
        in_specs=[
            pl.BlockSpec((1, bm, D), lambda i: (0, i, 0)),
            pl.BlockSpec((1, bm, D), lambda i: (1, i, 0)),
            pl.BlockSpec((bm, 1), lambda i: (i, 0)),
        ],
        out_specs=pl.BlockSpec((bm, D), lambda i: (i, 0)),
        out_shape=jax.ShapeDtypeStruct((n, D), jnp.float32),
    )(partials, partials, norm)


def _make_scatter(n_nodes, chunks_per_tile):
    align = NS * 8
    acc_rows = ((n_nodes + 1 + align - 1) // align) * align
    zrows = acc_rows // NS

    assert chunks_per_tile % SEG == 0

    @functools.partial(
        pl.kernel,
        out_type=jax.ShapeDtypeStruct((NC, acc_rows, D), jnp.float32),
        mesh=_mesh(),
        scratch_types=[
            pltpu.VMEM((SEG, CHUNK), jnp.int32),
            pltpu.VMEM((SEG, CHUNK), jnp.int32),
            pltpu.VMEM((CHUNK, D), jnp.float32),
            pltpu.VMEM((CHUNK, D), jnp.float32),
            pltpu.VMEM_SHARED((acc_rows, D), jnp.float32),
            pltpu.SemaphoreType.DMA,
            pltpu.SemaphoreType.DMA,
            pltpu.SemaphoreType.DMA,
            pltpu.SemaphoreType.DMA,
        ],
    )
    def scatter_kernel(hprime, main_idx, tail_idx, zeros_hbm, out, src_v,
                       dst_v, rows0, rows1, acc,
                       gsem0, gsem1, ssem0, ssem1):
        c = lax.axis_index("c")
        s = lax.axis_index("s")
        wid = c * NS + s
        trow = wid * chunks_per_tile
        is_tail = wid == NC * NS - 1
        rows = (rows0, rows1)
        gsem = (gsem0, gsem1)
        ssem = (ssem0, ssem1)
        pltpu.sync_copy(zeros_hbm, acc.at[pl.ds(s * zrows, zrows)])
        plsc.subcore_barrier()

        def wait_g(j, b):
            pltpu.make_async_copy(
                hprime.at[src_v.at[j]], rows[b], gsem[b]).wait()

        def issue_g(j, b):
            pltpu.async_copy(hprime.at[src_v.at[j]], rows[b], gsem[b])

        def issue_s(j, b):
            pltpu.async_copy(rows[b], acc.at[dst_v.at[j]], ssem[b], add=True)

        def wait_s(j, b):
            pltpu.make_async_copy(
                rows[b], acc.at[dst_v.at[j]], ssem[b]).wait()

        for sg in range(chunks_per_tile // SEG):
            base = sg * SEG

            @pl.when(is_tail)
            def _():
                pltpu.sync_copy(tail_idx.at[0, pl.ds(base, SEG)], src_v)
                pltpu.sync_copy(tail_idx.at[1, pl.ds(base, SEG)], dst_v)

            @pl.when(jnp.logical_not(is_tail))
            def _():
                pltpu.sync_copy(main_idx.at[0, pl.ds(trow + base, SEG)],
                                src_v)
                pltpu.sync_copy(main_idx.at[1, pl.ds(trow + base, SEG)],
                                dst_v)
            issue_g(0, 0)
            issue_g(1, 1)

            def step(i, carry):
                for b in range(2):
                    j = 2 * i + b
                    wait_g(j, b)
                    issue_s(j, b)
                    wait_s(j, b)
                    issue_g(j + 2, b)
                return carry

            lax.fori_loop(0, (SEG - 2) // 2, step, 0)
            for b in range(2):
                j = SEG - 2 + b
                wait_g(j, b)
                issue_s(j, b)
                wait_s(j, b)

        plsc.subcore_barrier()
        pltpu.sync_copy(acc.at[pl.ds(s * zrows, zrows)],
                        out.at[c, pl.ds(s * zrows, zrows)])

    return scatter_kernel, acc_rows


def kernel(h, edge_index, norm, W):
    n_nodes = h.shape[0]
    n_edges = edge_index.shape[1]

    hprime = _matmul_norm(h, W, norm)

    n_tiles = NC * NS
    chunks_per_tile = -(-n_edges // (n_tiles * CHUNK))
    chunks_per_tile = ((chunks_per_tile + SEG - 1) // SEG) * SEG

    scatter_kernel, acc_rows = _make_scatter(n_nodes, chunks_per_tile)

    assert n_edges % CHUNK == 0
    real_rows = n_edges // CHUNK
    rows_total = n_tiles * chunks_per_tile
    pad_rows = rows_total - real_rows
    assert 0 < pad_rows <= chunks_per_tile
    main_idx = edge_index.reshape(2, real_rows, CHUNK)
    pad = pad_rows * CHUNK
    spare = acc_rows - n_nodes
    pad_src = (jnp.arange(pad, dtype=jnp.int32) % n_nodes
               ).reshape(1, pad_rows, CHUNK)
    pad_dst = (n_nodes + jnp.arange(pad, dtype=jnp.int32) % spare
               ).reshape(1, pad_rows, CHUNK)
    pad_idx = jnp.concatenate([pad_src, pad_dst], axis=0)
    tail_real = chunks_per_tile - pad_rows
    tail_idx = jnp.concatenate(
        [main_idx[:, real_rows - tail_real:], pad_idx], axis=1)
    zeros = jnp.zeros((acc_rows // NS, D), jnp.float32)

    partials = scatter_kernel(hprime, main_idx, tail_idx, zeros)

    return _combine(partials, norm)


# speedup vs baseline: 1.0428x; 1.0428x over previous
import functools

import jax
import jax.numpy as jnp
from jax import lax
from jax.experimental import pallas as pl
from jax.experimental.pallas import tpu as pltpu
from jax.experimental.pallas import tpu_sc as plsc

NC = 2
NS = 16
CHUNK = 80
SEG = 64
D = 128


def _mesh():
    return plsc.VectorSubcoreMesh(
        core_axis_name="c", subcore_axis_name="s",
        num_cores=NC, num_subcores=NS)


def _mm_body(h_ref, w_ref, norm_ref, out_ref):
    out_ref[...] = (
        jnp.dot(h_ref[...], w_ref[...], preferred_element_type=jnp.float32)
        * norm_ref[...]
    )


def _matmul_norm(h, W, norm):
    n, d_in = h.shape
    d_out = W.shape[1]
    bm = 2000
    return pl.pallas_call(
        _mm_body,
        grid=(n // bm,),
        in_specs=[
            pl.BlockSpec((bm, d_in), lambda i: (i, 0)),
            pl.BlockSpec((d_in, d_out), lambda i: (0, 0)),
            pl.BlockSpec((bm, 1), lambda i: (i, 0)),
        ],
        out_specs=pl.BlockSpec((bm, d_out), lambda i: (i, 0)),
        out_shape=jax.ShapeDtypeStruct((n, d_out), jnp.float32),
    )(h, W, norm)


def _combine_body(a_ref, b_ref, norm_ref, out_ref):
    out_ref[...] = (a_ref[0] + b_ref[0]) * norm_ref[...]


def _combine(partials, norm):
    n, _ = norm.shape
    bm = 2000
    return pl.pallas_call(
        _combine_body,
        grid=(n // bm,),
        in_specs=[
            pl.BlockSpec((1, bm, D), lambda i: (0, i, 0)),
            pl.BlockSpec((1, bm, D), lambda i: (1, i, 0)),
            pl.BlockSpec((bm, 1), lambda i: (i, 0)),
        ],
        out_specs=pl.BlockSpec((bm, D), lambda i: (i, 0)),
        out_shape=jax.ShapeDtypeStruct((n, D), jnp.float32),
    )(partials, partials, norm)


def _make_scatter(n_nodes, chunks_per_tile):
    align = NS * 8
    acc_rows = ((n_nodes + 1 + align - 1) // align) * align
    zrows = acc_rows // NS

    assert chunks_per_tile % SEG == 0

    @functools.partial(
        pl.kernel,
        out_type=jax.ShapeDtypeStruct((NC, acc_rows, D), jnp.float32),
        mesh=_mesh(),
        scratch_types=[
            pltpu.VMEM((SEG, CHUNK), jnp.int32),
            pltpu.VMEM((SEG, CHUNK), jnp.int32),
            pltpu.VMEM((CHUNK, D), jnp.float32),
            pltpu.VMEM((CHUNK, D), jnp.float32),
            pltpu.VMEM((CHUNK, D), jnp.float32),
            pltpu.VMEM_SHARED((acc_rows, D), jnp.float32),
            pltpu.SemaphoreType.DMA,
            pltpu.SemaphoreType.DMA,
            pltpu.SemaphoreType.DMA,
            pltpu.SemaphoreType.DMA,
            pltpu.SemaphoreType.DMA,
            pltpu.SemaphoreType.DMA,
        ],
    )
    def scatter_kernel(hprime, main_idx, tail_idx, zeros_hbm, out, src_v,
                       dst_v, rows0, rows1, rows2, acc,
                       gsem0, gsem1, gsem2, ssem0, ssem1, ssem2):
        c = lax.axis_index("c")
        s = lax.axis_index("s")
        wid = c * NS + s
        trow = wid * chunks_per_tile
        is_tail = wid == NC * NS - 1
        rows = (rows0, rows1, rows2)
        gsem = (gsem0, gsem1, gsem2)
        ssem = (ssem0, ssem1, ssem2)
        pltpu.sync_copy(zeros_hbm, acc.at[pl.ds(s * zrows, zrows)])
        plsc.subcore_barrier()

        def wait_g(j, b):
            pltpu.make_async_copy(
                hprime.at[src_v.at[j]], rows[b], gsem[b]).wait()

        def issue_g(j, b):
            pltpu.async_copy(hprime.at[src_v.at[j]], rows[b], gsem[b])

        def issue_s(j, b):
            pltpu.async_copy(rows[b], acc.at[dst_v.at[j]], ssem[b], add=True)

        def wait_s(j, b):
            pltpu.make_async_copy(
                rows[b], acc.at[dst_v.at[j]], ssem[b]).wait()

        for sg in range(chunks_per_tile // SEG):
            base = sg * SEG

            @pl.when(is_tail)
            def _():
                pltpu.sync_copy(tail_idx.at[0, pl.ds(base, SEG)], src_v)
                pltpu.sync_copy(tail_idx.at[1, pl.ds(base, SEG)], dst_v)

            @pl.when(jnp.logical_not(is_tail))
            def _():
                pltpu.sync_copy(main_idx.at[0, pl.ds(trow + base, SEG)],
                                src_v)
                pltpu.sync_copy(main_idx.at[1, pl.ds(trow + base, SEG)],
                                dst_v)
            issue_g(0, 0)
            issue_g(1, 1)
            wait_g(0, 0)
            issue_s(0, 0)
            issue_g(2, 2)

            def step(i, carry):
                for k in range(3):
                    j = 3 * i + 1 + k
                    b = (1 + k) % 3
                    wait_g(j, b)
                    issue_s(j, b)
                    wait_s(j - 1, k % 3)
                    issue_g(j + 2, k % 3)
                return carry

            lax.fori_loop(0, (SEG - 4) // 3, step, 0)
            j = SEG - 3
            wait_g(j, j % 3)
            issue_s(j, j % 3)
            wait_s(j - 1, (j - 1) % 3)
            issue_g(j + 2, (j + 2) % 3)
            for j in (SEG - 2, SEG - 1):
                wait_g(j, j % 3)
                issue_s(j, j % 3)
                wait_s(j - 1, (j - 1) % 3)
            wait_s(SEG - 1, (SEG - 1) % 3)

        plsc.subcore_barrier()
        pltpu.sync_copy(acc.at[pl.ds(s * zrows, zrows)],
                        out.at[c, pl.ds(s * zrows, zrows)])

    return scatter_kernel, acc_rows


def kernel(h, edge_index, norm, W):
    n_nodes = h.shape[0]
    n_edges = edge_index.shape[1]

    hprime = _matmul_norm(h, W, norm)

    n_tiles = NC * NS
    chunks_per_tile = -(-n_edges // (n_tiles * CHUNK))
    chunks_per_tile = ((chunks_per_tile + SEG - 1) // SEG) * SEG

    scatter_kernel, acc_rows = _make_scatter(n_nodes, chunks_per_tile)

    assert n_edges % CHUNK == 0
    real_rows = n_edges // CHUNK
    rows_total = n_tiles * chunks_per_tile
    pad_rows = rows_total - real_rows
    assert 0 < pad_rows <= chunks_per_tile
    main_idx = edge_index.reshape(2, real_rows, CHUNK)
    pad = pad_rows * CHUNK
    spare = acc_rows - n_nodes
    pad_src = (jnp.arange(pad, dtype=jnp.int32) % n_nodes
               ).reshape(1, pad_rows, CHUNK)
    pad_dst = (n_nodes + jnp.arange(pad, dtype=jnp.int32) % spare
               ).reshape(1, pad_rows, CHUNK)
    pad_idx = jnp.concatenate([pad_src, pad_dst], axis=0)
    tail_real = chunks_per_tile - pad_rows
    tail_idx = jnp.concatenate(
        [main_idx[:, real_rows - tail_real:], pad_idx], axis=1)
    zeros = jnp.zeros((acc_rows // NS, D), jnp.float32)

    partials = scatter_kernel(hprime, main_idx, tail_idx, zeros)

    return _combine(partials, norm)
